# baseline (speedup 1.0000x reference)
import functools
import math

import jax
import jax.numpy as jnp
from jax import lax
from jax.experimental import pallas as pl
from jax.experimental.pallas import tpu as pltpu
from jax.experimental.pallas import tpu_sc as plsc

D_MODEL = 64
SCALE = math.sqrt(D_MODEL)

NUM_WORKERS = 32
CHUNK = 128
B_TOTAL = 4096 * 200
B_PER_W = B_TOTAL // NUM_WORKERS
CHUNKS_PER_W = B_PER_W // CHUNK


def _sc_embed(lut, xf):
    mesh = plsc.VectorSubcoreMesh(core_axis_name="c", subcore_axis_name="s")
    info = plsc.get_sparse_core_info()
    nc = info.num_cores

    @functools.partial(
        pl.kernel,
        mesh=mesh,
        out_type=jax.ShapeDtypeStruct((NUM_WORKERS, CHUNKS_PER_W, CHUNK, D_MODEL),
                                      jnp.float32),
        scratch_types=[
            pltpu.VMEM((CHUNK,), jnp.int32),
            pltpu.VMEM((CHUNK, D_MODEL), jnp.float32),
            pltpu.SemaphoreType.DMA,
        ],
        compiler_params=pltpu.CompilerParams(use_tc_tiling_on_sc=False),
    )
    def k(lut_hbm, idx_hbm, out_hbm, idx_v, rows_v, sem):
        wid = lax.axis_index("s") * nc + lax.axis_index("c")

        def chunk_body(j, carry):
            pltpu.sync_copy(idx_hbm.at[wid, j], idx_v)
            pltpu.async_copy(lut_hbm.at[idx_v], rows_v, sem).wait()

            def row_body(r, c):
                for q in range(D_MODEL // 16):
                    sl = pl.ds(q * 16, 16)
                    rows_v[r, sl] = rows_v[r, sl] * SCALE
                return c

            lax.fori_loop(0, CHUNK, row_body, 0)
            pltpu.sync_copy(rows_v, out_hbm.at[wid, j])
            return carry

        lax.fori_loop(0, CHUNKS_PER_W, chunk_body, 0)

    return k(lut, xf)


def kernel(x, lut):
    xf = jnp.reshape(x, (NUM_WORKERS, CHUNKS_PER_W, CHUNK)).astype(jnp.int32)
    out = _sc_embed(lut, xf)
    return jnp.reshape(out, (4096, 200, D_MODEL))


# speedup vs baseline: 1.2687x; 1.2687x over previous
import functools
import math

import jax
import jax.numpy as jnp
from jax import lax
from jax.experimental import pallas as pl
from jax.experimental.pallas import tpu as pltpu
from jax.experimental.pallas import tpu_sc as plsc

D_MODEL = 64
SCALE = math.sqrt(D_MODEL)

NUM_WORKERS = 32
CHUNK = 128
B_TOTAL = 4096 * 200
B_PER_W = B_TOTAL // NUM_WORKERS
CHUNKS_PER_W = B_PER_W // CHUNK
NBUF = 4


def _sc_embed(lut, xf):
    mesh = plsc.VectorSubcoreMesh(core_axis_name="c", subcore_axis_name="s")
    info = plsc.get_sparse_core_info()
    nc = info.num_cores

    @functools.partial(
        pl.kernel,
        mesh=mesh,
        out_type=jax.ShapeDtypeStruct((NUM_WORKERS, CHUNKS_PER_W, CHUNK, D_MODEL),
                                      jnp.float32),
        scratch_types=[
            pltpu.VMEM((CHUNKS_PER_W, CHUNK), jnp.int32),
            pltpu.VMEM((NBUF, CHUNK, D_MODEL), jnp.float32),
            pltpu.VMEM((NBUF, CHUNK, D_MODEL), jnp.float32),
            pltpu.SemaphoreType.DMA((NBUF,)),
            pltpu.SemaphoreType.DMA((NBUF,)),
        ],
        compiler_params=pltpu.CompilerParams(use_tc_tiling_on_sc=False),
    )
    def k(lut_hbm, idx_hbm, out_hbm, idx_v, gbuf, sbuf, gsem, osem):
        wid = lax.axis_index("s") * nc + lax.axis_index("c")
        pltpu.sync_copy(idx_hbm.at[wid], idx_v)

        def gather_start(s, j):
            pltpu.async_copy(lut_hbm.at[idx_v.at[j]], gbuf.at[s], gsem.at[s])

        def gather_wait(s, j):
            pltpu.make_async_copy(
                lut_hbm.at[idx_v.at[j]], gbuf.at[s], gsem.at[s]).wait()

        def out_start(s, j):
            pltpu.async_copy(sbuf.at[s], out_hbm.at[wid, j], osem.at[s])

        def out_wait(s, j):
            pltpu.make_async_copy(
                sbuf.at[s], out_hbm.at[wid, j], osem.at[s]).wait()

        for s in range(NBUF):
            gather_start(s, s)

        def body(i, carry):
            j0 = i * NBUF
            for s in range(NBUF):
                j = j0 + s

                @pl.when(j >= NBUF)
                def _():
                    out_wait(s, j - NBUF)

                gather_wait(s, j)

                def srow(r, c):
                    for q in range(D_MODEL // 16):
                        sl = pl.ds(q * 16, 16)
                        sbuf[s, r, sl] = gbuf[s, r, sl] * SCALE
                    return c

                lax.fori_loop(0, CHUNK, srow, 0)

                @pl.when(j + NBUF < CHUNKS_PER_W)
                def _():
                    gather_start(s, j + NBUF)

                out_start(s, j)
            return carry

        lax.fori_loop(0, CHUNKS_PER_W // NBUF, body, 0)

        for s in range(NBUF):
            out_wait(s, CHUNKS_PER_W - NBUF + s)

    return k(lut, xf)


def kernel(x, lut):
    xf = jnp.reshape(x, (NUM_WORKERS, CHUNKS_PER_W, CHUNK)).astype(jnp.int32)
    out = _sc_embed(lut, xf)
    return jnp.reshape(out, (4096, 200, D_MODEL))


# speedup vs baseline: 1.2704x; 1.0014x over previous
import functools
import math

import jax
import jax.numpy as jnp
from jax import lax
from jax.experimental import pallas as pl
from jax.experimental.pallas import tpu as pltpu
from jax.experimental.pallas import tpu_sc as plsc

D_MODEL = 64
SCALE = math.sqrt(D_MODEL)

NUM_WORKERS = 32
N_BATCH = 4096
N_TOK = 200
B_PER_W = N_BATCH // NUM_WORKERS
NBUF = 4
SPLIT = 104


def _sc_embed(lut, x):
    mesh = plsc.VectorSubcoreMesh(core_axis_name="c", subcore_axis_name="s")
    info = plsc.get_sparse_core_info()
    nc = info.num_cores

    @functools.partial(
        pl.kernel,
        mesh=mesh,
        out_type=jax.ShapeDtypeStruct((N_BATCH, N_TOK, D_MODEL), jnp.float32),
        scratch_types=[
            pltpu.VMEM((B_PER_W, N_TOK), jnp.int32),
            pltpu.VMEM((NBUF, N_TOK, D_MODEL), jnp.float32),
            pltpu.VMEM((NBUF, N_TOK, D_MODEL), jnp.float32),
            pltpu.SemaphoreType.DMA((NBUF,)),
            pltpu.SemaphoreType.DMA((NBUF,)),
        ],
        compiler_params=pltpu.CompilerParams(use_tc_tiling_on_sc=False),
    )
    def k(lut_hbm, idx_hbm, out_hbm, idx_v, gbuf, sbuf, gsem, osem):
        wid = lax.axis_index("s") * nc + lax.axis_index("c")
        b0 = wid * B_PER_W
        pltpu.sync_copy(idx_hbm.at[pl.ds(b0, B_PER_W)], idx_v)

        def gather_start(s, i):
            pltpu.async_copy(lut_hbm.at[idx_v.at[i, pl.ds(0, SPLIT)]],
                             gbuf.at[s, pl.ds(0, SPLIT)], gsem.at[s])
            pltpu.async_copy(lut_hbm.at[idx_v.at[i, pl.ds(SPLIT, N_TOK - SPLIT)]],
                             gbuf.at[s, pl.ds(SPLIT, N_TOK - SPLIT)], gsem.at[s])

        def gather_wait(s, i):
            pltpu.make_async_copy(
                lut_hbm.at[idx_v.at[i, pl.ds(0, SPLIT)]],
                gbuf.at[s, pl.ds(0, SPLIT)], gsem.at[s]).wait()
            pltpu.make_async_copy(
                lut_hbm.at[idx_v.at[i, pl.ds(SPLIT, N_TOK - SPLIT)]],
                gbuf.at[s, pl.ds(SPLIT, N_TOK - SPLIT)], gsem.at[s]).wait()

        def out_start(s, i):
            pltpu.async_copy(sbuf.at[s], out_hbm.at[b0 + i], osem.at[s])

        def out_wait(s, i):
            pltpu.make_async_copy(sbuf.at[s], out_hbm.at[b0 + i],
                                  osem.at[s]).wait()

        for s in range(NBUF):
            gather_start(s, s)

        def body(it, carry):
            i0 = it * NBUF
            for s in range(NBUF):
                i = i0 + s

                @pl.when(i >= NBUF)
                def _():
                    out_wait(s, i - NBUF)

                gather_wait(s, i)

                def srow(r, c):
                    for q in range(D_MODEL // 16):
                        sl = pl.ds(q * 16, 16)
                        sbuf[s, r, sl] = gbuf[s, r, sl] * SCALE
                    return c

                lax.fori_loop(0, N_TOK, srow, 0)

                @pl.when(i + NBUF < B_PER_W)
                def _():
                    gather_start(s, i + NBUF)

                out_start(s, i)
            return carry

        lax.fori_loop(0, B_PER_W // NBUF, body, 0)

        for s in range(NBUF):
            out_wait(s, B_PER_W - NBUF + s)

    return k(lut, x)


def kernel(x, lut):
    return _sc_embed(lut, x)


# speedup vs baseline: 1.4064x; 1.1071x over previous
import functools
import math

import jax
import jax.numpy as jnp
from jax import lax
from jax.experimental import pallas as pl
from jax.experimental.pallas import tpu as pltpu
from jax.experimental.pallas import tpu_sc as plsc

D_MODEL = 64
SCALE = math.sqrt(D_MODEL)

VOCAB = 1000000
NUM_WORKERS = 32
N_BATCH = 4096
N_TOK = 200
B_PER_W = N_BATCH // NUM_WORKERS
NBUF = 4
SPLIT = 104

PACK_W = 2048


def _tc_pack_lut(lut):
    lut_t = jnp.transpose(lut)

    def body(in_ref, out_ref):
        x = in_ref[...]
        y = jnp.transpose(x)
        y3 = jnp.reshape(y, (PACK_W // 2, 2, D_MODEL))
        out_ref[...] = jnp.concatenate([y3[:, 0, :], y3[:, 1, :]], axis=-1)

    grid = pl.cdiv(VOCAB, PACK_W)
    packed = pl.pallas_call(
        body,
        grid=(grid,),
        in_specs=[pl.BlockSpec((D_MODEL, PACK_W), lambda i: (0, i))],
        out_specs=pl.BlockSpec((PACK_W // 2, 2 * D_MODEL), lambda i: (i, 0)),
        out_shape=jax.ShapeDtypeStruct((VOCAB // 2, 2 * D_MODEL), jnp.float32),
    )(lut_t)
    return jnp.reshape(jnp.reshape(packed, (VOCAB * D_MODEL,)),
                       (VOCAB, D_MODEL))


def _tc_finish(d2):
    def body(in_ref, out_ref):
        out_ref[...] = jnp.transpose(in_ref[...])

    out_t = pl.pallas_call(
        body,
        grid=(N_BATCH // 512, (N_TOK * D_MODEL) // 512),
        in_specs=[pl.BlockSpec((512, 512), lambda i, j: (i, j))],
        out_specs=pl.BlockSpec((512, 512), lambda i, j: (j, i)),
        out_shape=jax.ShapeDtypeStruct((N_TOK * D_MODEL, N_BATCH),
                                       jnp.float32),
    )(d2)
    out3 = jnp.reshape(out_t, (N_TOK, D_MODEL, N_BATCH))
    return jnp.transpose(out3, (2, 0, 1))


def _sc_embed(lut, x):
    mesh = plsc.VectorSubcoreMesh(core_axis_name="c", subcore_axis_name="s")
    info = plsc.get_sparse_core_info()
    nc = info.num_cores

    @functools.partial(
        pl.kernel,
        mesh=mesh,
        out_type=jax.ShapeDtypeStruct((N_BATCH, N_TOK * D_MODEL), jnp.float32),
        scratch_types=[
            pltpu.VMEM((B_PER_W, N_TOK), jnp.int32),
            pltpu.VMEM((NBUF, N_TOK, D_MODEL), jnp.float32),
            pltpu.VMEM((NBUF, N_TOK * D_MODEL), jnp.float32),
            pltpu.SemaphoreType.DMA((NBUF,)),
            pltpu.SemaphoreType.DMA((NBUF,)),
        ],
        compiler_params=pltpu.CompilerParams(use_tc_tiling_on_sc=False),
    )
    def k(lut_hbm, idx_hbm, out_hbm, idx_v, gbuf, sbuf, gsem, osem):
        wid = lax.axis_index("s") * nc + lax.axis_index("c")
        b0 = wid * B_PER_W
        pltpu.sync_copy(idx_hbm.at[pl.ds(b0, B_PER_W)], idx_v)

        def gather_start(s, i):
            pltpu.async_copy(lut_hbm.at[idx_v.at[i, pl.ds(0, SPLIT)]],
                             gbuf.at[s, pl.ds(0, SPLIT)], gsem.at[s])
            pltpu.async_copy(lut_hbm.at[idx_v.at[i, pl.ds(SPLIT, N_TOK - SPLIT)]],
                             gbuf.at[s, pl.ds(SPLIT, N_TOK - SPLIT)], gsem.at[s])

        def gather_wait(s, i):
            pltpu.make_async_copy(
                lut_hbm.at[idx_v.at[i, pl.ds(0, SPLIT)]],
                gbuf.at[s, pl.ds(0, SPLIT)], gsem.at[s]).wait()
            pltpu.make_async_copy(
                lut_hbm.at[idx_v.at[i, pl.ds(SPLIT, N_TOK - SPLIT)]],
                gbuf.at[s, pl.ds(SPLIT, N_TOK - SPLIT)], gsem.at[s]).wait()

        def out_start(s, i):
            pltpu.async_copy(sbuf.at[s], out_hbm.at[b0 + i], osem.at[s])

        def out_wait(s, i):
            pltpu.make_async_copy(sbuf.at[s], out_hbm.at[b0 + i],
                                  osem.at[s]).wait()

        for s in range(NBUF):
            gather_start(s, s)

        def body(it, carry):
            i0 = it * NBUF
            for s in range(NBUF):
                i = i0 + s

                @pl.when(i >= NBUF)
                def _():
                    out_wait(s, i - NBUF)

                gather_wait(s, i)

                def srow(r, c):
                    for q in range(D_MODEL // 16):
                        sbuf[s, pl.ds(r * D_MODEL + q * 16, 16)] = (
                            gbuf[s, r, pl.ds(q * 16, 16)] * SCALE)
                    return c

                lax.fori_loop(0, N_TOK, srow, 0)

                @pl.when(i + NBUF < B_PER_W)
                def _():
                    gather_start(s, i + NBUF)

                out_start(s, i)
            return carry

        lax.fori_loop(0, B_PER_W // NBUF, body, 0)

        for s in range(NBUF):
            out_wait(s, B_PER_W - NBUF + s)

    return k(lut, x)


def kernel(x, lut):
    lut_rm = _tc_pack_lut(lut)
    dense = _sc_embed(lut_rm, x)
    return _tc_finish(dense)


# speedup vs baseline: 1.5164x; 1.0782x over previous
import functools
import math

import jax
import jax.numpy as jnp
from jax import lax
from jax.experimental import pallas as pl
from jax.experimental.pallas import tpu as pltpu
from jax.experimental.pallas import tpu_sc as plsc

D_MODEL = 64
SCALE = math.sqrt(D_MODEL)

VOCAB = 1000000
NUM_WORKERS = 32
N_BATCH = 4096
N_TOK = 200
B_PER_W = N_BATCH // NUM_WORKERS
NBUF = 4
SPLIT = 104

PACK_W = 4096


def _tc_pack_lut(lut):
    lut_t = jnp.transpose(lut)
    eye = jnp.eye(D_MODEL, dtype=jnp.float32)

    def body(in_ref, eye_ref, out_ref):
        x = in_ref[...]
        y = jax.lax.dot_general(x, eye_ref[...], (((0,), (0,)), ((), ())),
                                preferred_element_type=jnp.float32)
        y3 = jnp.reshape(y, (PACK_W // 2, 2, D_MODEL))
        out_ref[...] = jnp.concatenate([y3[:, 0, :], y3[:, 1, :]], axis=-1)

    grid = pl.cdiv(VOCAB, PACK_W)
    packed = pl.pallas_call(
        body,
        grid=(grid,),
        in_specs=[pl.BlockSpec((D_MODEL, PACK_W), lambda i: (0, i)),
                  pl.BlockSpec((D_MODEL, D_MODEL), lambda i: (0, 0))],
        out_specs=pl.BlockSpec((PACK_W // 2, 2 * D_MODEL), lambda i: (i, 0)),
        out_shape=jax.ShapeDtypeStruct((VOCAB // 2, 2 * D_MODEL), jnp.float32),
    )(lut_t, eye)
    return jnp.reshape(jnp.reshape(packed, (VOCAB * D_MODEL,)),
                       (VOCAB, D_MODEL))


def _tc_finish(d2):
    def body(in_ref, out_ref):
        out_ref[...] = jnp.transpose(in_ref[...])

    out_t = pl.pallas_call(
        body,
        grid=(N_BATCH // 512, (N_TOK * D_MODEL) // 512),
        in_specs=[pl.BlockSpec((512, 512), lambda i, j: (i, j))],
        out_specs=pl.BlockSpec((512, 512), lambda i, j: (j, i)),
        out_shape=jax.ShapeDtypeStruct((N_TOK * D_MODEL, N_BATCH),
                                       jnp.float32),
    )(d2)
    out3 = jnp.reshape(out_t, (N_TOK, D_MODEL, N_BATCH))
    return jnp.transpose(out3, (2, 0, 1))


def _sc_embed(lut, x):
    mesh = plsc.VectorSubcoreMesh(core_axis_name="c", subcore_axis_name="s")
    info = plsc.get_sparse_core_info()
    nc = info.num_cores

    @functools.partial(
        pl.kernel,
        mesh=mesh,
        out_type=jax.ShapeDtypeStruct((N_BATCH, N_TOK * D_MODEL), jnp.float32),
        scratch_types=[
            pltpu.VMEM((B_PER_W, N_TOK), jnp.int32),
            pltpu.VMEM((NBUF, N_TOK, D_MODEL), jnp.float32),
            pltpu.VMEM((NBUF, N_TOK * D_MODEL), jnp.float32),
            pltpu.SemaphoreType.DMA((NBUF,)),
            pltpu.SemaphoreType.DMA((NBUF,)),
        ],
        compiler_params=pltpu.CompilerParams(use_tc_tiling_on_sc=False),
    )
    def k(lut_hbm, idx_hbm, out_hbm, idx_v, gbuf, sbuf, gsem, osem):
        wid = lax.axis_index("s") * nc + lax.axis_index("c")
        b0 = wid * B_PER_W
        pltpu.sync_copy(idx_hbm.at[pl.ds(b0, B_PER_W)], idx_v)

        def gather_start(s, i):
            pltpu.async_copy(lut_hbm.at[idx_v.at[i, pl.ds(0, SPLIT)]],
                             gbuf.at[s, pl.ds(0, SPLIT)], gsem.at[s])
            pltpu.async_copy(lut_hbm.at[idx_v.at[i, pl.ds(SPLIT, N_TOK - SPLIT)]],
                             gbuf.at[s, pl.ds(SPLIT, N_TOK - SPLIT)], gsem.at[s])

        def gather_wait(s, i):
            pltpu.make_async_copy(
                lut_hbm.at[idx_v.at[i, pl.ds(0, SPLIT)]],
                gbuf.at[s, pl.ds(0, SPLIT)], gsem.at[s]).wait()
            pltpu.make_async_copy(
                lut_hbm.at[idx_v.at[i, pl.ds(SPLIT, N_TOK - SPLIT)]],
                gbuf.at[s, pl.ds(SPLIT, N_TOK - SPLIT)], gsem.at[s]).wait()

        def out_start(s, i):
            pltpu.async_copy(sbuf.at[s], out_hbm.at[b0 + i], osem.at[s])

        def out_wait(s, i):
            pltpu.make_async_copy(sbuf.at[s], out_hbm.at[b0 + i],
                                  osem.at[s]).wait()

        for s in range(NBUF):
            gather_start(s, s)

        def body(it, carry):
            i0 = it * NBUF
            for s in range(NBUF):
                i = i0 + s

                @pl.when(i >= NBUF)
                def _():
                    out_wait(s, i - NBUF)

                gather_wait(s, i)

                def srow(r, c):
                    for q in range(D_MODEL // 16):
                        sbuf[s, pl.ds(r * D_MODEL + q * 16, 16)] = (
                            gbuf[s, r, pl.ds(q * 16, 16)] * SCALE)
                    return c

                lax.fori_loop(0, N_TOK, srow, 0)

                @pl.when(i + NBUF < B_PER_W)
                def _():
                    gather_start(s, i + NBUF)

                out_start(s, i)
            return carry

        lax.fori_loop(0, B_PER_W // NBUF, body, 0)

        for s in range(NBUF):
            out_wait(s, B_PER_W - NBUF + s)

    return k(lut, x)


def kernel(x, lut):
    lut_rm = _tc_pack_lut(lut)
    dense = _sc_embed(lut_rm, x)
    return _tc_finish(dense)
